# baseline (speedup 1.0000x reference)
import functools

import jax
import jax.numpy as jnp
from jax import lax
from jax.experimental import pallas as pl
from jax.experimental.pallas import tpu as pltpu
from jax.experimental.pallas import tpu_sc as plsc

N_NODES = 10000
N_EDGES = 160000
HIDDEN = 256
EPS = 1e-5

NUM_CORES = 2
NUM_SUBCORES = 16
NT = NUM_CORES * NUM_SUBCORES
TROWS = 313
PAD_ROWS = 8
ACC_ROWS = TROWS + PAD_ROWS
LAST_TROWS = N_NODES - (NT - 1) * TROWS
PAD_ROWS = 8
ACC_ROWS = TROWS + PAD_ROWS
ECHUNK = 2000
N_CHUNKS = N_EDGES // ECHUNK
GROUPS = ECHUNK // 16
GB = 32
CAP = 6464
C_FLUSH = CAP - 2 * ECHUNK - 64

MM_BLOCK = 1000


def _mm_body(x_ref, wt_ref, o_ref):
    o_ref[:] = jnp.maximum(
        jnp.dot(x_ref[:], wt_ref[:], preferred_element_type=jnp.float32), 0.0)


def _matmul_relu(x, wt):
    return pl.pallas_call(
        _mm_body,
        grid=(N_NODES // MM_BLOCK,),
        in_specs=[
            pl.BlockSpec((MM_BLOCK, HIDDEN), lambda i: (i, 0)),
            pl.BlockSpec((HIDDEN, HIDDEN), lambda i: (0, 0)),
        ],
        out_specs=pl.BlockSpec((MM_BLOCK, HIDDEN), lambda i: (i, 0)),
        out_shape=jax.ShapeDtypeStruct((N_NODES, HIDDEN), jnp.float32),
    )(x, wt)


def _sc_body(m_hbm, dst_hbm, src_hbm, zeros_hbm, agg_hbm,
             dstA, srcA, dstB, srcB, csrc_v, cldst_v,
             rowsA, rowsB, acc_v, baseb_v, semA, semB, semGA, semGB):
    c = lax.axis_index("c")
    s = lax.axis_index("s")
    w = c * NUM_SUBCORES + s
    lo = w * TROWS
    hi = jnp.minimum(lo + TROWS, N_NODES)
    iota16 = lax.iota(jnp.int32, 16)

    pltpu.sync_copy(zeros_hbm, acc_v)

    def fire_chunk(i, db, sb, sem):
        pltpu.async_copy(dst_hbm.at[pl.ds(i * ECHUNK, ECHUNK)], db, sem)
        pltpu.async_copy(src_hbm.at[pl.ds(i * ECHUNK, ECHUNK)], sb, sem)

    def drain_chunk(db, sb, sem):
        pltpu.make_async_copy(dst_hbm.at[pl.ds(0, ECHUNK)], db, sem).wait()
        pltpu.make_async_copy(src_hbm.at[pl.ds(0, ECHUNK)], sb, sem).wait()

    def fire_gather(gb, rows, sem):
        pltpu.async_copy(m_hbm.at[csrc_v.at[pl.ds(gb * GB, GB)]], rows, sem)

    def drain_gather(gb, rows, sem):
        pltpu.make_async_copy(
            m_hbm.at[csrc_v.at[pl.ds(gb * GB, GB)]], rows, sem).wait()

    def apply_batch(gb, rows):
        for sg in range(GB // 16):
            bases16 = cldst_v[pl.ds(gb * GB + sg * 16, 16)] * HIDDEN
            bases = [bases16[k] for k in range(16)]

            @plsc.parallel_loop(0, HIDDEN, 16, unroll=2)
            def _(cc):
                for k in range(16):
                    plsc.addupdate(acc_v.at[pl.ds(bases[k] + cc, 16)],
                                   rows[sg * 16 + k, pl.ds(cc, 16)])

    def apply_all(cnt, cnt_vec):
        trash = TROWS + (iota16 & (PAD_ROWS - 1))
        plsc.store_scatter(csrc_v, [cnt_vec + iota16], iota16)
        plsc.store_scatter(csrc_v, [cnt_vec + 16 + iota16], iota16)
        plsc.store_scatter(cldst_v, [cnt_vec + iota16], trash)
        plsc.store_scatter(cldst_v, [cnt_vec + 16 + iota16], trash)
        ngb = (cnt + GB - 1) // GB

        @pl.when(ngb > 0)
        def _():
            fire_gather(0, rowsA, semGA)

        @pl.loop(0, ngb, step=2)
        def _(g):
            @pl.when(g + 1 < ngb)
            def _():
                fire_gather(g + 1, rowsB, semGB)

            drain_gather(g, rowsA, semGA)
            apply_batch(g, rowsA)

            @pl.when(g + 2 < ngb)
            def _():
                fire_gather(g + 2, rowsA, semGA)

            @pl.when(g + 1 < ngb)
            def _():
                drain_gather(g + 1, rowsB, semGB)
                apply_batch(g + 1, rowsB)

    def scan_chunk(db, sb, cnt_vec):
        @plsc.parallel_loop(0, GROUPS, 1, unroll=25, carry=cnt_vec)
        def final(j, cnt_vec):
            d = db[pl.ds(j * 16, 16)]
            sv = sb[pl.ds(j * 16, 16)]
            msk = (d >= lo) & (d < hi)
            pc = plsc.all_reduce_population_count(msk)
            ones = jnp.where(msk, jnp.int32(1), jnp.int32(0))
            pos = cnt_vec + plsc.cumsum(ones) - 1
            plsc.store_scatter(csrc_v, [pos], sv, mask=msk)
            plsc.store_scatter(cldst_v, [pos], d - lo, mask=msk)
            return cnt_vec + pc

        return final

    fire_chunk(0, dstA, srcA, semA)

    def pair(t, cnt_vec):
        i = 2 * t
        drain_chunk(dstA, srcA, semA)
        fire_chunk(i + 1, dstB, srcB, semB)
        cnt_vec = scan_chunk(dstA, srcA, cnt_vec)
        drain_chunk(dstB, srcB, semB)
        fire_chunk(jnp.minimum(i + 2, N_CHUNKS - 1), dstA, srcA, semA)
        cnt_vec = scan_chunk(dstB, srcB, cnt_vec)

        cnt = cnt_vec[0]
        flush = cnt >= C_FLUSH

        @pl.when(flush)
        def _():
            apply_all(cnt, cnt_vec)

        return jnp.where(flush, jnp.zeros_like(cnt_vec), cnt_vec)

    cnt_vec = lax.fori_loop(
        0, N_CHUNKS // 2, pair, jnp.zeros((16,), jnp.int32))
    drain_chunk(dstA, srcA, semA)

    apply_all(cnt_vec[0], cnt_vec)

    @pl.when(w < NT - 1)
    def _():
        pltpu.sync_copy(acc_v.at[pl.ds(0, TROWS * HIDDEN)],
                        agg_hbm.at[pl.ds(lo * HIDDEN, TROWS * HIDDEN)])

    @pl.when(w == NT - 1)
    def _():
        pltpu.sync_copy(acc_v.at[pl.ds(0, LAST_TROWS * HIDDEN)],
                        agg_hbm.at[pl.ds(lo * HIDDEN, LAST_TROWS * HIDDEN)])


def _scatter_add(m, dst, src):
    zeros = jnp.zeros((ACC_ROWS * HIDDEN,), jnp.float32)
    mesh = plsc.VectorSubcoreMesh(core_axis_name="c", subcore_axis_name="s")
    fn = functools.partial(
        pl.kernel,
        mesh=mesh,
        compiler_params=pltpu.CompilerParams(needs_layout_passes=False),
        out_type=jax.ShapeDtypeStruct((N_NODES * HIDDEN,), jnp.float32),
        scratch_types=[
            pltpu.VMEM((ECHUNK,), jnp.int32),
            pltpu.VMEM((ECHUNK,), jnp.int32),
            pltpu.VMEM((ECHUNK,), jnp.int32),
            pltpu.VMEM((ECHUNK,), jnp.int32),
            pltpu.VMEM((CAP,), jnp.int32),
            pltpu.VMEM((CAP,), jnp.int32),
            pltpu.VMEM((GB, HIDDEN), jnp.float32),
            pltpu.VMEM((GB, HIDDEN), jnp.float32),
            pltpu.VMEM((ACC_ROWS * HIDDEN,), jnp.float32),
            pltpu.VMEM((32,), jnp.int32),
            pltpu.SemaphoreType.DMA,
            pltpu.SemaphoreType.DMA,
            pltpu.SemaphoreType.DMA,
            pltpu.SemaphoreType.DMA,
        ],
    )(_sc_body)
    return fn(m, dst, src, zeros)


def _norm_body(x_ref, agg_ref, g_ref, o_ref):
    h = x_ref[:] + agg_ref[:]
    inv = lax.rsqrt(jnp.mean(h * h, axis=-1, keepdims=True) + EPS)
    o_ref[:] = h * inv * g_ref[:]


def _residual_norm(x, agg, g):
    return pl.pallas_call(
        _norm_body,
        grid=(N_NODES // MM_BLOCK,),
        in_specs=[
            pl.BlockSpec((MM_BLOCK, HIDDEN), lambda i: (i, 0)),
            pl.BlockSpec((MM_BLOCK, HIDDEN), lambda i: (i, 0)),
            pl.BlockSpec((HIDDEN,), lambda i: (0,)),
        ],
        out_specs=pl.BlockSpec((MM_BLOCK, HIDDEN), lambda i: (i, 0)),
        out_shape=jax.ShapeDtypeStruct((N_NODES, HIDDEN), jnp.float32),
    )(x, agg, g)


def kernel(x, edge_index, W, g):
    dst = edge_index[0]
    src = edge_index[1]
    m = _matmul_relu(x, W.T)
    agg = _scatter_add(m, dst, src).reshape(N_NODES, HIDDEN)
    return _residual_norm(x, agg, g)


# speedup vs baseline: 1.2486x; 1.2486x over previous
import functools

import jax
import jax.numpy as jnp
from jax import lax
from jax.experimental import pallas as pl
from jax.experimental.pallas import tpu as pltpu
from jax.experimental.pallas import tpu_sc as plsc

N_NODES = 10000
N_EDGES = 160000
HIDDEN = 256
EPS = 1e-5

NUM_CORES = 2
NUM_SUBCORES = 16
NT = NUM_CORES * NUM_SUBCORES
TROWS = 313
PAD_ROWS = 8
ACC_ROWS = TROWS + PAD_ROWS
LAST_TROWS = N_NODES - (NT - 1) * TROWS
PAD_ROWS = 8
ACC_ROWS = TROWS + PAD_ROWS
ECHUNK = 2000
N_CHUNKS = N_EDGES // ECHUNK
GROUPS = ECHUNK // 16
GB = 32
CAP = 6464
C_FLUSH = CAP - 2 * ECHUNK - 64

MM_BLOCK = 1000


def _mm_body(x_ref, wt_ref, o_ref):
    o_ref[:] = jnp.maximum(
        jnp.dot(x_ref[:], wt_ref[:], preferred_element_type=jnp.float32), 0.0)


def _matmul_relu(x, wt):
    return pl.pallas_call(
        _mm_body,
        grid=(N_NODES // MM_BLOCK,),
        in_specs=[
            pl.BlockSpec((MM_BLOCK, HIDDEN), lambda i: (i, 0)),
            pl.BlockSpec((HIDDEN, HIDDEN), lambda i: (0, 0)),
        ],
        out_specs=pl.BlockSpec((MM_BLOCK, HIDDEN), lambda i: (i, 0)),
        out_shape=jax.ShapeDtypeStruct((N_NODES, HIDDEN), jnp.float32),
    )(x, wt)


def _sc_body(m_hbm, dst_hbm, src_hbm, zeros_hbm, agg_hbm,
             dstA, srcA, dstB, srcB, csrc_v, cldst_v,
             rowsA, rowsB, acc_v, baseb_v, semA, semB, semGA, semGB):
    c = lax.axis_index("c")
    s = lax.axis_index("s")
    w = c * NUM_SUBCORES + s
    lo = w * TROWS
    hi = jnp.minimum(lo + TROWS, N_NODES)
    iota16 = lax.iota(jnp.int32, 16)

    pltpu.sync_copy(zeros_hbm, acc_v)

    def fire_chunk(i, db, sb, sem):
        pltpu.async_copy(dst_hbm.at[pl.ds(i * ECHUNK, ECHUNK)], db, sem)
        pltpu.async_copy(src_hbm.at[pl.ds(i * ECHUNK, ECHUNK)], sb, sem)

    def drain_chunk(db, sb, sem):
        pltpu.make_async_copy(dst_hbm.at[pl.ds(0, ECHUNK)], db, sem).wait()
        pltpu.make_async_copy(src_hbm.at[pl.ds(0, ECHUNK)], sb, sem).wait()

    def fire_gather(gb, rows, sem):
        pltpu.async_copy(m_hbm.at[csrc_v.at[pl.ds(gb * GB, GB)]], rows, sem)

    def drain_gather(gb, rows, sem):
        pltpu.make_async_copy(
            m_hbm.at[csrc_v.at[pl.ds(gb * GB, GB)]], rows, sem).wait()

    def apply_batch(gb, rows):
        for sg in range(GB // 16):
            bases16 = cldst_v[pl.ds(gb * GB + sg * 16, 16)] * HIDDEN
            bases = [bases16[k] for k in range(16)]

            @plsc.parallel_loop(0, HIDDEN, 16, unroll=2)
            def _(cc):
                for k in range(16):
                    plsc.addupdate(acc_v.at[pl.ds(bases[k] + cc, 16)],
                                   rows[sg * 16 + k, pl.ds(cc, 16)])

    def apply_all(cnt, cnt_vec):
        trash = TROWS + (iota16 & (PAD_ROWS - 1))
        plsc.store_scatter(csrc_v, [cnt_vec + iota16], iota16)
        plsc.store_scatter(csrc_v, [cnt_vec + 16 + iota16], iota16)
        plsc.store_scatter(cldst_v, [cnt_vec + iota16], trash)
        plsc.store_scatter(cldst_v, [cnt_vec + 16 + iota16], trash)
        ngb = (cnt + GB - 1) // GB

        @pl.when(ngb > 0)
        def _():
            fire_gather(0, rowsA, semGA)

        @pl.loop(0, ngb, step=2)
        def _(g):
            @pl.when(g + 1 < ngb)
            def _():
                fire_gather(g + 1, rowsB, semGB)

            drain_gather(g, rowsA, semGA)
            apply_batch(g, rowsA)

            @pl.when(g + 2 < ngb)
            def _():
                fire_gather(g + 2, rowsA, semGA)

            @pl.when(g + 1 < ngb)
            def _():
                drain_gather(g + 1, rowsB, semGB)
                apply_batch(g + 1, rowsB)

    def scan_chunk(db, sb, cnt_vec):
        @plsc.parallel_loop(0, GROUPS, 1, unroll=5, carry=cnt_vec)
        def final(j, cnt_vec):
            d = db[pl.ds(j * 16, 16)]
            sv = sb[pl.ds(j * 16, 16)]
            msk = (d >= lo) & (d < hi)
            pc = plsc.all_reduce_population_count(msk)
            ones = jnp.where(msk, jnp.int32(1), jnp.int32(0))
            pos = cnt_vec + plsc.cumsum(ones) - 1
            plsc.store_scatter(csrc_v, [pos], sv, mask=msk)
            plsc.store_scatter(cldst_v, [pos], d - lo, mask=msk)
            return cnt_vec + pc

        return final

    fire_chunk(0, dstA, srcA, semA)

    def pair(t, cnt_vec):
        i = 2 * t
        drain_chunk(dstA, srcA, semA)
        fire_chunk(i + 1, dstB, srcB, semB)
        cnt_vec = scan_chunk(dstA, srcA, cnt_vec)
        drain_chunk(dstB, srcB, semB)
        fire_chunk(jnp.minimum(i + 2, N_CHUNKS - 1), dstA, srcA, semA)
        cnt_vec = scan_chunk(dstB, srcB, cnt_vec)

        cnt = cnt_vec[0]
        flush = cnt >= C_FLUSH

        @pl.when(flush)
        def _():
            apply_all(cnt, cnt_vec)

        return jnp.where(flush, jnp.zeros_like(cnt_vec), cnt_vec)

    cnt_vec = lax.fori_loop(
        0, N_CHUNKS // 2, pair, jnp.zeros((16,), jnp.int32))
    drain_chunk(dstA, srcA, semA)

    apply_all(cnt_vec[0], cnt_vec)

    @pl.when(w < NT - 1)
    def _():
        pltpu.sync_copy(acc_v.at[pl.ds(0, TROWS * HIDDEN)],
                        agg_hbm.at[pl.ds(lo * HIDDEN, TROWS * HIDDEN)])

    @pl.when(w == NT - 1)
    def _():
        pltpu.sync_copy(acc_v.at[pl.ds(0, LAST_TROWS * HIDDEN)],
                        agg_hbm.at[pl.ds(lo * HIDDEN, LAST_TROWS * HIDDEN)])


def _scatter_add(m, dst, src):
    zeros = jnp.zeros((ACC_ROWS * HIDDEN,), jnp.float32)
    mesh = plsc.VectorSubcoreMesh(core_axis_name="c", subcore_axis_name="s")
    fn = functools.partial(
        pl.kernel,
        mesh=mesh,
        compiler_params=pltpu.CompilerParams(needs_layout_passes=False),
        out_type=jax.ShapeDtypeStruct((N_NODES * HIDDEN,), jnp.float32),
        scratch_types=[
            pltpu.VMEM((ECHUNK,), jnp.int32),
            pltpu.VMEM((ECHUNK,), jnp.int32),
            pltpu.VMEM((ECHUNK,), jnp.int32),
            pltpu.VMEM((ECHUNK,), jnp.int32),
            pltpu.VMEM((CAP,), jnp.int32),
            pltpu.VMEM((CAP,), jnp.int32),
            pltpu.VMEM((GB, HIDDEN), jnp.float32),
            pltpu.VMEM((GB, HIDDEN), jnp.float32),
            pltpu.VMEM((ACC_ROWS * HIDDEN,), jnp.float32),
            pltpu.VMEM((32,), jnp.int32),
            pltpu.SemaphoreType.DMA,
            pltpu.SemaphoreType.DMA,
            pltpu.SemaphoreType.DMA,
            pltpu.SemaphoreType.DMA,
        ],
    )(_sc_body)
    return fn(m, dst, src, zeros)


def _norm_body(x_ref, agg_ref, g_ref, o_ref):
    h = x_ref[:] + agg_ref[:]
    inv = lax.rsqrt(jnp.mean(h * h, axis=-1, keepdims=True) + EPS)
    o_ref[:] = h * inv * g_ref[:]


def _residual_norm(x, agg, g):
    return pl.pallas_call(
        _norm_body,
        grid=(N_NODES // MM_BLOCK,),
        in_specs=[
            pl.BlockSpec((MM_BLOCK, HIDDEN), lambda i: (i, 0)),
            pl.BlockSpec((MM_BLOCK, HIDDEN), lambda i: (i, 0)),
            pl.BlockSpec((HIDDEN,), lambda i: (0,)),
        ],
        out_specs=pl.BlockSpec((MM_BLOCK, HIDDEN), lambda i: (i, 0)),
        out_shape=jax.ShapeDtypeStruct((N_NODES, HIDDEN), jnp.float32),
    )(x, agg, g)


def kernel(x, edge_index, W, g):
    dst = edge_index[0]
    src = edge_index[1]
    m = _matmul_relu(x, W.T)
    agg = _scatter_add(m, dst, src).reshape(N_NODES, HIDDEN)
    return _residual_norm(x, agg, g)
